# baseline (speedup 1.0000x reference)
import functools

import jax
import jax.numpy as jnp
from jax import lax
from jax.experimental import pallas as pl
from jax.experimental.pallas import tpu as pltpu
from jax.experimental.pallas import tpu_sc as plsc

VOCAB = 100000
D = 64
DP = 128
B = 16384
NT = 2 * B
NC = 2
NS = 16
NW = NC * NS
SPT = NT // NW
SB = 128
NSB = SPT // SB
BPW = B // NW
CHUNK = 128
NCHUNK = BPW // CHUNK
LANES = 16
WIN = 1024
WSHIFT = 10
VROWS = 8 * VOCAB

_mesh = plsc.VectorSubcoreMesh(core_axis_name="c", subcore_axis_name="s")


@functools.partial(
    pl.kernel,
    mesh=_mesh,
    out_type=jax.ShapeDtypeStruct((NT, DP), jnp.float32),
    scratch_types=[
        pltpu.VMEM((SPT,), jnp.int32),
        pltpu.VMEM((64, WIN), jnp.float32),
        pltpu.VMEM((SB, DP), jnp.float32),
        pltpu.VMEM((SB, DP), jnp.float32),
        pltpu.SemaphoreType.DMA,
    ],
    compiler_params=pltpu.CompilerParams(use_tc_tiling_on_sc=True,
                                         needs_layout_passes=False),
)
def _extract(t_h, s_h, out_h, s_v, tbuf, rbuf0, rbuf1, wsem):
    wid = lax.axis_index("s") * NC + lax.axis_index("c")
    base = wid * SPT
    pltpu.sync_copy(s_h.at[pl.ds(base, SPT)], s_v)

    def lane_scalar(vec, lane):
        return lax.squeeze(lax.slice(vec, (lane,), (lane + 1,)), (0,))

    cvecs = [q * LANES + lax.iota(jnp.int32, LANES) for q in range(4)]
    rbufs = (rbuf0, rbuf1)
    writes = []
    prev_rg = jnp.int32(-1)
    for sb in range(NSB):
        rbuf = rbufs[sb % 2]
        if len(writes) >= 2:
            writes[sb - 2].wait()

        def group_body(g, prev, rbuf=rbuf, sb=sb):
            sv = s_v[pl.ds(sb * SB + g * LANES, LANES)]
            for lane in range(LANES):
                sflat = lane_scalar(sv, lane)
                win = lax.shift_right_logical(sflat, WSHIFT)

                @pl.when(win != prev)
                def _():
                    wstart = lax.min(win * WIN, VROWS - WIN)
                    pltpu.sync_copy(t_h.at[:, pl.ds(wstart, WIN)], tbuf)

                wstart = lax.min(win * WIN, VROWS - WIN)
                colsplat = jnp.full((LANES,), sflat - wstart, jnp.int32)
                row = g * LANES + lane
                for q in range(4):
                    vals = plsc.load_gather(tbuf, [cvecs[q], colsplat])
                    rbuf[row, pl.ds(q * LANES, LANES)] = vals
                prev = win
            return prev

        prev_rg = lax.fori_loop(0, SB // LANES, group_body, prev_rg)
        writes.append(
            pltpu.async_copy(rbuf, out_h.at[pl.ds(base + sb * SB, SB)], wsem))
    for w in writes[-2:]:
        w.wait()


@functools.partial(
    pl.kernel,
    mesh=_mesh,
    out_type=tuple(jax.ShapeDtypeStruct((B, DP), jnp.float32) for _ in range(4)),
    scratch_types=[
        pltpu.VMEM((BPW,), jnp.int32),
        pltpu.VMEM((8, CHUNK), jnp.int32),
        pltpu.VMEM((BPW, DP), jnp.float32),
        pltpu.SemaphoreType.DMA,
        pltpu.SemaphoreType.DMA,
    ],
    compiler_params=pltpu.CompilerParams(use_tc_tiling_on_sc=True),
)
def _permute(rw_h, rc_h, iwa_h, iwb_h, ica_h, icb_h,
             owa, owb, oca, ocb, iv, i2d, buf, gsem, wsem):
    wid = lax.axis_index("s") * NC + lax.axis_index("c")
    base = wid * BPW
    for r_h, ih, out in ((rw_h, iwa_h, owa), (rw_h, iwb_h, owb),
                         (rc_h, ica_h, oca), (rc_h, icb_h, ocb)):
        pltpu.sync_copy(ih.at[pl.ds(base, BPW)], iv)
        for k in range(BPW // LANES):
            i2d[k // 8, pl.ds((k % 8) * LANES, LANES)] = iv[pl.ds(k * LANES, LANES)]
        g = [
            pltpu.async_copy(r_h.at[i2d.at[j]],
                             buf.at[pl.ds(j * CHUNK, CHUNK)], gsem)
            for j in range(NCHUNK)
        ]
        for c in g:
            c.wait()
        pltpu.async_copy(buf, out.at[pl.ds(base, BPW)], wsem).wait()


def kernel(word_idx, ctx_same, ctx_other, variety_a, variety_b,
           word_table, ctx_table):
    i32 = jnp.int32
    wi = word_idx.astype(i32)
    va = variety_a.astype(i32)
    vb = variety_b.astype(i32)
    fw = jnp.concatenate([va * VOCAB + wi, vb * VOCAB + wi])
    fc = jnp.concatenate([va * VOCAB + ctx_same.astype(i32),
                          vb * VOCAB + ctx_other.astype(i32)])
    pw = jnp.argsort(fw).astype(i32)
    pc = jnp.argsort(fc).astype(i32)
    sw = jnp.take(fw, pw)
    sc = jnp.take(fc, pc)
    slots = jnp.arange(NT, dtype=i32)
    invw = jnp.zeros((NT,), i32).at[pw].set(slots)
    invc = jnp.zeros((NT,), i32).at[pc].set(slots)

    rows_w = _extract(word_table.T, sw)
    rows_c = _extract(ctx_table.T, sc)
    word_emb_a, word_emb_b, ctx_emb_a, ctx_emb_b = _permute(
        rows_w, rows_c, invw[:B], invw[B:], invc[:B], invc[B:])
    return (word_emb_a[:, :D], ctx_emb_a[:, :D],
            ctx_emb_b[:, :D], word_emb_b[:, :D])


# speedup vs baseline: 1.0122x; 1.0122x over previous
import functools

import jax
import jax.numpy as jnp
from jax import lax
from jax.experimental import pallas as pl
from jax.experimental.pallas import tpu as pltpu
from jax.experimental.pallas import tpu_sc as plsc

VOCAB = 100000
D = 64
DP = 128
B = 16384
NT = 2 * B
NC = 2
NS = 16
NW = NC * NS
SPT = NT // NW
SB = 128
NSB = SPT // SB
BPW = B // NW
CHUNK = 128
NCHUNK = BPW // CHUNK
LANES = 16
WIN = 1024
WSHIFT = 10
VROWS = 8 * VOCAB

_mesh = plsc.VectorSubcoreMesh(core_axis_name="c", subcore_axis_name="s")


@functools.partial(
    pl.kernel,
    mesh=_mesh,
    out_type=jax.ShapeDtypeStruct((NT, DP), jnp.float32),
    scratch_types=[
        pltpu.VMEM((SPT,), jnp.int32),
        pltpu.VMEM((64, WIN), jnp.float32),
        pltpu.VMEM((SB, DP), jnp.float32),
        pltpu.VMEM((SB, DP), jnp.float32),
        pltpu.SemaphoreType.DMA,
    ],
    compiler_params=pltpu.CompilerParams(use_tc_tiling_on_sc=True,
                                         needs_layout_passes=False),
)
def _extract(t_h, s_h, out_h, s_v, tbuf, rbuf0, rbuf1, wsem):
    wid = lax.axis_index("s") * NC + lax.axis_index("c")
    base = wid * SPT
    pltpu.sync_copy(s_h.at[pl.ds(base, SPT)], s_v)

    def lane_scalar(vec, lane):
        return lax.squeeze(lax.slice(vec, (lane,), (lane + 1,)), (0,))

    cvecs = [q * LANES + lax.iota(jnp.int32, LANES) for q in range(4)]
    rbufs = (rbuf0, rbuf1)
    writes = []
    prev_rg = jnp.int32(-1)
    for sb in range(NSB):
        rbuf = rbufs[sb % 2]
        if len(writes) >= 2:
            writes[sb - 2].wait()

        def group_body(g, prev, rbuf=rbuf, sb=sb):
            sv = s_v[pl.ds(sb * SB + g * LANES, LANES)]
            for lane in range(LANES):
                sflat = lane_scalar(sv, lane)
                win = lax.shift_right_logical(sflat, WSHIFT)

                @pl.when(win != prev)
                def _():
                    wstart = lax.min(win * WIN, VROWS - WIN)
                    pltpu.sync_copy(t_h.at[:, pl.ds(wstart, WIN)], tbuf)

                wstart = lax.min(win * WIN, VROWS - WIN)
                colsplat = jnp.full((LANES,), sflat - wstart, jnp.int32)
                row = g * LANES + lane
                for q in range(4):
                    vals = plsc.load_gather(tbuf, [cvecs[q], colsplat])
                    rbuf[row, pl.ds(q * LANES, LANES)] = vals
                prev = win
            return prev

        prev_rg = lax.fori_loop(0, SB // LANES, group_body, prev_rg)
        writes.append(
            pltpu.async_copy(rbuf, out_h.at[pl.ds(base + sb * SB, SB)], wsem))
    for w in writes[-2:]:
        w.wait()


@functools.partial(
    pl.kernel,
    mesh=_mesh,
    out_type=tuple(jax.ShapeDtypeStruct((B, DP), jnp.float32) for _ in range(2)),
    scratch_types=[
        pltpu.VMEM((BPW,), jnp.int32),
        pltpu.VMEM((8, CHUNK), jnp.int32),
        pltpu.VMEM((BPW, DP), jnp.float32),
        pltpu.SemaphoreType.DMA,
        pltpu.SemaphoreType.DMA,
    ],
    compiler_params=pltpu.CompilerParams(use_tc_tiling_on_sc=True),
)
def _permute(r_h, ia_h, ib_h, oa, ob, iv, i2d, buf, gsem, wsem):
    wid = lax.axis_index("s") * NC + lax.axis_index("c")
    base = wid * BPW
    for ih, out in ((ia_h, oa), (ib_h, ob)):
        pltpu.sync_copy(ih.at[pl.ds(base, BPW)], iv)
        for k in range(BPW // LANES):
            i2d[k // 8, pl.ds((k % 8) * LANES, LANES)] = iv[pl.ds(k * LANES, LANES)]
        g = [
            pltpu.async_copy(r_h.at[i2d.at[j]],
                             buf.at[pl.ds(j * CHUNK, CHUNK)], gsem)
            for j in range(NCHUNK)
        ]
        for c in g:
            c.wait()
        pltpu.async_copy(buf, out.at[pl.ds(base, BPW)], wsem).wait()


def kernel(word_idx, ctx_same, ctx_other, variety_a, variety_b,
           word_table, ctx_table):
    i32 = jnp.int32
    wi = word_idx.astype(i32)
    va = variety_a.astype(i32)
    vb = variety_b.astype(i32)
    fw = jnp.concatenate([va * VOCAB + wi, vb * VOCAB + wi])
    fc = jnp.concatenate([va * VOCAB + ctx_same.astype(i32),
                          vb * VOCAB + ctx_other.astype(i32)])
    pw = jnp.argsort(fw).astype(i32)
    pc = jnp.argsort(fc).astype(i32)
    sw = jnp.take(fw, pw)
    sc = jnp.take(fc, pc)
    slots = jnp.arange(NT, dtype=i32)
    invw = jnp.zeros((NT,), i32).at[pw].set(slots)
    invc = jnp.zeros((NT,), i32).at[pc].set(slots)

    rows_w = _extract(word_table.T, sw)
    rows_c = _extract(ctx_table.T, sc)
    word_emb_a, word_emb_b = _permute(rows_w, invw[:B], invw[B:])
    ctx_emb_a, ctx_emb_b = _permute(rows_c, invc[:B], invc[B:])
    return (word_emb_a[:, :D], ctx_emb_a[:, :D],
            ctx_emb_b[:, :D], word_emb_b[:, :D])
